# Initial kernel scaffold; baseline (speedup 1.0000x reference)
import jax
import jax.numpy as jnp
from jax.experimental import pallas as pl


def kernel(x, embed):
    raise NotImplementedError("write your pallas kernel here")


# speedup vs baseline: 2.6129x; 2.6129x over previous
import functools

import jax
import jax.numpy as jnp
from jax import lax
from jax.experimental import pallas as pl
from jax.experimental.pallas import tpu as pltpu
from jax.experimental.pallas import tpu_sc as plsc

N_TOK = 36864
C = 8192
D = 64
N_BLK = 256


def _dist_argmin_body(x_ref, et_ref, idx_ref):
    x = x_ref[...]
    et = et_ref[...]
    cross = jnp.dot(x, et, preferred_element_type=jnp.float32)
    x_sq = jnp.sum(x * x, axis=-1, keepdims=True)
    e_sq = jnp.sum(et * et, axis=0, keepdims=True)
    dist = (x_sq + e_sq) - 2.0 * cross
    m = jnp.min(dist, axis=-1, keepdims=True)
    iota = lax.broadcasted_iota(jnp.int32, dist.shape, 1)
    idx = jnp.min(jnp.where(dist <= m, iota, jnp.int32(C)), axis=-1,
                  keepdims=True)
    idx_ref[...] = idx


def _dist_argmin(x2d, embed_t):
    grid = (N_TOK // N_BLK,)
    return pl.pallas_call(
        _dist_argmin_body,
        grid=grid,
        in_specs=[
            pl.BlockSpec((N_BLK, D), lambda i: (i, 0)),
            pl.BlockSpec((D, C), lambda i: (0, 0)),
        ],
        out_specs=pl.BlockSpec((N_BLK, 1), lambda i: (i, 0)),
        out_shape=jax.ShapeDtypeStruct((N_TOK, 1), jnp.int32),
    )(x2d, embed_t)


_SC_INFO = plsc.get_sparse_core_info()
_NW = _SC_INFO.num_cores * _SC_INFO.num_subcores
_B_PER_W = N_TOK // _NW


@functools.partial(
    pl.kernel,
    out_type=jax.ShapeDtypeStruct((N_TOK, D), jnp.float32),
    mesh=plsc.VectorSubcoreMesh(core_axis_name="c", subcore_axis_name="s"),
    compiler_params=pltpu.CompilerParams(use_tc_tiling_on_sc=False),
    scratch_types=[
        pltpu.VMEM((_B_PER_W,), jnp.int32),
        pltpu.VMEM((_B_PER_W, D), jnp.float32),
        pltpu.SemaphoreType.DMA,
    ],
)
def _sc_gather(table_hbm, idx_hbm, out_hbm, idx_v, rows_v, sem):
    wid = lax.axis_index("s") * _SC_INFO.num_cores + lax.axis_index("c")
    base = wid * _B_PER_W
    pltpu.sync_copy(idx_hbm.at[pl.ds(base, _B_PER_W)], idx_v)
    pltpu.async_copy(table_hbm.at[idx_v], rows_v, sem).wait()
    pltpu.sync_copy(rows_v, out_hbm.at[pl.ds(base, _B_PER_W)])


def kernel(x, embed):
    x2d = x[0].astype(jnp.float32)
    table = embed[0].astype(jnp.float32)
    embed_t = table.T
    idx = _dist_argmin(x2d, embed_t)
    out = _sc_gather(table, idx.reshape(N_TOK))
    return out[None]


# speedup vs baseline: 3.5903x; 1.3741x over previous
import functools

import jax
import jax.numpy as jnp
from jax import lax
from jax.experimental import pallas as pl
from jax.experimental.pallas import tpu as pltpu
from jax.experimental.pallas import tpu_sc as plsc

N_TOK = 36864
C = 8192
D = 64
N_BLK = 256


LANES = 128
CHUNK = 512


def _dist_argmin_body(x_ref, et_ref, idx_ref, esq_ref):
    @pl.when(pl.program_id(0) == 0)
    def _():
        et = et_ref[...]
        esq_ref[...] = jnp.sum(et * et, axis=0, keepdims=True)

    x = x_ref[...]
    x_sq = jnp.sum(x * x, axis=-1, keepdims=True)
    xx = x + x
    best_v = jnp.full((N_BLK, LANES), jnp.inf, jnp.float32)
    best_t = jnp.zeros((N_BLK, LANES), jnp.float32)
    for k in range(C // CHUNK):
        cs = slice(k * CHUNK, (k + 1) * CHUNK)
        twocross = jnp.dot(xx, et_ref[:, cs],
                           preferred_element_type=jnp.float32)
        d = (x_sq + esq_ref[:, cs]) - twocross
        for j in range(CHUNK // LANES):
            dv = d[:, j * LANES:(j + 1) * LANES]
            t = jnp.float32(k * (CHUNK // LANES) + j)
            upd = dv < best_v
            best_v = jnp.where(upd, dv, best_v)
            best_t = jnp.where(upd, t, best_t)
    lane = lax.broadcasted_iota(jnp.int32, (N_BLK, LANES), 1).astype(jnp.float32)
    cfull = best_t * jnp.float32(LANES) + lane
    m = jnp.min(best_v, axis=-1, keepdims=True)
    idxf = jnp.min(jnp.where(best_v <= m, cfull, jnp.float32(C)),
                   axis=-1, keepdims=True)
    idx_ref[...] = idxf.astype(jnp.int32)


def _dist_argmin(x2d, embed_t):
    grid = (N_TOK // N_BLK,)
    return pl.pallas_call(
        _dist_argmin_body,
        grid=grid,
        in_specs=[
            pl.BlockSpec((N_BLK, D), lambda i: (i, 0)),
            pl.BlockSpec((D, C), lambda i: (0, 0)),
        ],
        out_specs=pl.BlockSpec((N_BLK, 1), lambda i: (i, 0)),
        out_shape=jax.ShapeDtypeStruct((N_TOK, 1), jnp.int32),
        scratch_shapes=[pltpu.VMEM((1, C), jnp.float32)],
    )(x2d, embed_t)


@functools.cache
def _sc_gather_fn():
    info = plsc.get_sparse_core_info()
    nc = info.num_cores
    nw = nc * info.num_subcores
    b_per_w = N_TOK // nw

    @functools.partial(
        pl.kernel,
        out_type=jax.ShapeDtypeStruct((N_TOK, D), jnp.float32),
        mesh=plsc.VectorSubcoreMesh(core_axis_name="c", subcore_axis_name="s"),
        compiler_params=pltpu.CompilerParams(use_tc_tiling_on_sc=False),
        scratch_types=[
            pltpu.VMEM((b_per_w,), jnp.int32),
            pltpu.VMEM((b_per_w, D), jnp.float32),
            pltpu.SemaphoreType.DMA,
        ],
    )
    def _sc_gather(table_hbm, idx_hbm, out_hbm, idx_v, rows_v, sem):
        wid = lax.axis_index("s") * nc + lax.axis_index("c")
        base = wid * b_per_w
        pltpu.sync_copy(idx_hbm.at[pl.ds(base, b_per_w)], idx_v)
        pltpu.async_copy(table_hbm.at[idx_v], rows_v, sem).wait()
        pltpu.sync_copy(rows_v, out_hbm.at[pl.ds(base, b_per_w)])

    return _sc_gather


def kernel(x, embed):
    x2d = x[0].astype(jnp.float32)
    table = embed[0].astype(jnp.float32)
    embed_t = table.T
    idx = _dist_argmin(x2d, embed_t)
    out = _sc_gather_fn()(table, idx.reshape(N_TOK))
    return out[None]


# speedup vs baseline: 3.9838x; 1.1096x over previous
import functools

import jax
import jax.numpy as jnp
from jax import lax
from jax.experimental import pallas as pl
from jax.experimental.pallas import tpu as pltpu
from jax.experimental.pallas import tpu_sc as plsc

N_TOK = 36864
C = 8192
D = 64
N_BLK = 2048


LANES = 128
CHUNK = 512


def _dist_argmin_body(x_ref, et_ref, idx_ref, esq_ref):
    @pl.when(pl.program_id(0) == 0)
    def _():
        et = et_ref[...]
        esq_ref[...] = jnp.sum(et * et, axis=0, keepdims=True)

    x = x_ref[...]
    x_sq = jnp.sum(x * x, axis=-1, keepdims=True)
    xx = x + x
    best_v = jnp.full((N_BLK, LANES), jnp.inf, jnp.float32)
    best_t = jnp.zeros((N_BLK, LANES), jnp.float32)
    for k in range(C // CHUNK):
        cs = slice(k * CHUNK, (k + 1) * CHUNK)
        twocross = jnp.dot(xx, et_ref[:, cs],
                           preferred_element_type=jnp.float32)
        d = (x_sq + esq_ref[:, cs]) - twocross
        for j in range(CHUNK // LANES):
            dv = d[:, j * LANES:(j + 1) * LANES]
            t = jnp.float32(k * (CHUNK // LANES) + j)
            upd = dv < best_v
            best_v = jnp.where(upd, dv, best_v)
            best_t = jnp.where(upd, t, best_t)
    lane = lax.broadcasted_iota(jnp.int32, (N_BLK, LANES), 1).astype(jnp.float32)
    cfull = best_t * jnp.float32(LANES) + lane
    m = jnp.min(best_v, axis=-1, keepdims=True)
    idxf = jnp.min(jnp.where(best_v <= m, cfull, jnp.float32(C)),
                   axis=-1, keepdims=True)
    idx_ref[...] = idxf.astype(jnp.int32)


def _dist_argmin(x2d, embed_t):
    grid = (N_TOK // N_BLK,)
    return pl.pallas_call(
        _dist_argmin_body,
        grid=grid,
        in_specs=[
            pl.BlockSpec((N_BLK, D), lambda i: (i, 0)),
            pl.BlockSpec((D, C), lambda i: (0, 0)),
        ],
        out_specs=pl.BlockSpec((N_BLK, 1), lambda i: (i, 0)),
        out_shape=jax.ShapeDtypeStruct((N_TOK, 1), jnp.int32),
        scratch_shapes=[pltpu.VMEM((1, C), jnp.float32)],
    )(x2d, embed_t)


@functools.cache
def _sc_gather_fn():
    info = plsc.get_sparse_core_info()
    nc = info.num_cores
    nw = nc * info.num_subcores
    b_per_w = N_TOK // nw

    @functools.partial(
        pl.kernel,
        out_type=jax.ShapeDtypeStruct((N_TOK, D), jnp.float32),
        mesh=plsc.VectorSubcoreMesh(core_axis_name="c", subcore_axis_name="s"),
        compiler_params=pltpu.CompilerParams(use_tc_tiling_on_sc=False),
        scratch_types=[
            pltpu.VMEM((b_per_w,), jnp.int32),
            pltpu.VMEM((b_per_w, D), jnp.float32),
            pltpu.SemaphoreType.DMA,
        ],
    )
    def _sc_gather(table_hbm, idx_hbm, out_hbm, idx_v, rows_v, sem):
        wid = lax.axis_index("s") * nc + lax.axis_index("c")
        base = wid * b_per_w
        pltpu.sync_copy(idx_hbm.at[pl.ds(base, b_per_w)], idx_v)
        pltpu.async_copy(table_hbm.at[idx_v], rows_v, sem).wait()
        pltpu.sync_copy(rows_v, out_hbm.at[pl.ds(base, b_per_w)])

    return _sc_gather


def kernel(x, embed):
    x2d = x[0].astype(jnp.float32)
    table = embed[0].astype(jnp.float32)
    embed_t = table.T
    idx = _dist_argmin(x2d, embed_t)
    out = _sc_gather_fn()(table, idx.reshape(N_TOK))
    return out[None]


# speedup vs baseline: 3.9919x; 1.0020x over previous
import functools

import jax
import jax.numpy as jnp
from jax import lax
from jax.experimental import pallas as pl
from jax.experimental.pallas import tpu as pltpu
from jax.experimental.pallas import tpu_sc as plsc

N_TOK = 36864
C = 8192
D = 64
N_BLK = 2048


LANES = 128
CHUNK = 512


def _dist_argmin_body(x_ref, e_ref, idx_ref, esq_ref, et_ref):
    @pl.when(pl.program_id(0) == 0)
    def _():
        et = e_ref[...].T
        et_ref[...] = et
        esq_ref[...] = jnp.sum(et * et, axis=0, keepdims=True)

    x = x_ref[...]
    x_sq = jnp.sum(x * x, axis=-1, keepdims=True)
    xx = x + x
    best_v = jnp.full((N_BLK, LANES), jnp.inf, jnp.float32)
    best_t = jnp.zeros((N_BLK, LANES), jnp.float32)
    for k in range(C // CHUNK):
        cs = slice(k * CHUNK, (k + 1) * CHUNK)
        twocross = jnp.dot(xx, et_ref[:, cs],
                           preferred_element_type=jnp.float32)
        d = (x_sq + esq_ref[:, cs]) - twocross
        for j in range(CHUNK // LANES):
            dv = d[:, j * LANES:(j + 1) * LANES]
            t = jnp.float32(k * (CHUNK // LANES) + j)
            upd = dv < best_v
            best_v = jnp.where(upd, dv, best_v)
            best_t = jnp.where(upd, t, best_t)
    lane = lax.broadcasted_iota(jnp.int32, (N_BLK, LANES), 1).astype(jnp.float32)
    cfull = best_t * jnp.float32(LANES) + lane
    m = jnp.min(best_v, axis=-1, keepdims=True)
    idxf = jnp.min(jnp.where(best_v <= m, cfull, jnp.float32(C)),
                   axis=-1, keepdims=True)
    idx_ref[...] = idxf.astype(jnp.int32)


def _dist_argmin(x2d, table):
    grid = (N_TOK // N_BLK,)
    return pl.pallas_call(
        _dist_argmin_body,
        grid=grid,
        in_specs=[
            pl.BlockSpec((N_BLK, D), lambda i: (i, 0)),
            pl.BlockSpec((C, D), lambda i: (0, 0)),
        ],
        out_specs=pl.BlockSpec((N_BLK, 1), lambda i: (i, 0)),
        out_shape=jax.ShapeDtypeStruct((N_TOK, 1), jnp.int32),
        scratch_shapes=[pltpu.VMEM((1, C), jnp.float32),
                        pltpu.VMEM((D, C), jnp.float32)],
    )(x2d, table)


@functools.cache
def _sc_gather_fn():
    info = plsc.get_sparse_core_info()
    nc = info.num_cores
    nw = nc * info.num_subcores
    b_per_w = N_TOK // nw

    @functools.partial(
        pl.kernel,
        out_type=jax.ShapeDtypeStruct((N_TOK, D), jnp.float32),
        mesh=plsc.VectorSubcoreMesh(core_axis_name="c", subcore_axis_name="s"),
        compiler_params=pltpu.CompilerParams(use_tc_tiling_on_sc=False),
        scratch_types=[
            pltpu.VMEM((b_per_w,), jnp.int32),
            pltpu.VMEM((b_per_w, D), jnp.float32),
            pltpu.SemaphoreType.DMA,
        ],
    )
    def _sc_gather(table_hbm, idx_hbm, out_hbm, idx_v, rows_v, sem):
        wid = lax.axis_index("s") * nc + lax.axis_index("c")
        base = wid * b_per_w
        pltpu.sync_copy(idx_hbm.at[pl.ds(base, b_per_w)], idx_v)
        pltpu.async_copy(table_hbm.at[idx_v], rows_v, sem).wait()
        pltpu.sync_copy(rows_v, out_hbm.at[pl.ds(base, b_per_w)])

    return _sc_gather


def kernel(x, embed):
    x2d = x[0].astype(jnp.float32)
    table = embed[0].astype(jnp.float32)
    idx = _dist_argmin(x2d, table)
    out = _sc_gather_fn()(table, idx.reshape(N_TOK))
    return out[None]
